# baseline (speedup 1.0000x reference)
import functools

import jax
import jax.numpy as jnp
from jax import lax
from jax.experimental import pallas as pl
from jax.experimental.pallas import tpu as pltpu
from jax.experimental.pallas import tpu_sc as plsc

_NC, _NS, _L = 2, 16, 16
_NW = _NC * _NS


def _make_sc_rev(R, F):
    rows_per_w = R // _NW
    CH = 16
    n_chunks = rows_per_w // CH
    n_vec = F // _L
    mesh = plsc.VectorSubcoreMesh(core_axis_name="c", subcore_axis_name="s")

    @functools.partial(
        pl.kernel,
        mesh=mesh,
        out_type=jax.ShapeDtypeStruct((R, F), jnp.float32),
        scratch_types=[
            pltpu.VMEM((CH, F), jnp.float32),
            pltpu.VMEM((CH, F), jnp.float32),
        ],
    )
    def _sc_rev(x_hbm, o_hbm, buf_in, buf_out):
        wid = lax.axis_index("s") * _NC + lax.axis_index("c")
        base = wid * rows_per_w

        def chunk_body(ci, carry):
            row0 = base + ci * CH
            pltpu.sync_copy(x_hbm.at[pl.ds(row0, CH)], buf_in)

            def row_body(r, carry2):
                def c_body(co, carry3):
                    for u in range(8):
                        c = co * 8 + u
                        v = buf_in[r, pl.ds(c * _L, _L)]
                        buf_out[r, pl.ds((n_vec - 1) * _L - c * _L, _L)] = (
                            lax.rev(v, (0,))
                        )
                    return carry3

                lax.fori_loop(0, n_vec // 8, c_body, 0)
                return carry2

            lax.fori_loop(0, CH, row_body, 0)
            pltpu.sync_copy(buf_out, o_hbm.at[pl.ds(row0, CH)])
            return carry

        lax.fori_loop(0, n_chunks, chunk_body, 0)

    return _sc_rev


def kernel(inputs):
    B, S, F = inputs.shape
    R = B * S
    x = inputs.reshape(R, F)
    out = _make_sc_rev(R, F)(x)
    return out.reshape(B, S, F)
